# Initial kernel scaffold; baseline (speedup 1.0000x reference)
import jax
import jax.numpy as jnp
from jax.experimental import pallas as pl


def kernel(x, node_depth, edge_index, batch, type_emb, attr_emb, depth_emb, W0, as0, ad0, b0, g0, be0, W1, as1, ad1, b1, g1, be1, W2, as2, ad2, b2, g2, be2, W3, as3, ad3, b3, g3, be3, token_W, token_b):
    raise NotImplementedError("write your pallas kernel here")


# speedup vs baseline: 36.3390x; 36.3390x over previous
import functools

import jax
import jax.numpy as jnp
from jax import lax
from jax.experimental import pallas as pl
from jax.experimental.pallas import tpu as pltpu
from jax.experimental.pallas import tpu_sc as plsc

N = 10000
HID = 128
NGRAPHS = 128
MAX_DEPTH = 20
HEADS = (8, 8, 8, 1)

NC = 2
NS = 16
NW = NC * NS

NPAD = 10016
ROWS_PER_TILE = NPAD // NS

NEMB = 10240
EMB_ROWS_PER_W = NEMB // NW
EMB_BATCH = 80
EMB_ITERS = EMB_ROWS_PER_W // EMB_BATCH

EB = 128


def _mesh():
    return plsc.VectorSubcoreMesh(core_axis_name="c", subcore_axis_name="s")


_SC_PARAMS = pltpu.CompilerParams(use_tc_tiling_on_sc=False)


def _take16(x, idx):
    dnums = lax.GatherDimensionNumbers(
        offset_dims=(), collapsed_slice_dims=(0,), start_index_map=(0,))
    return lax.gather(x, idx[:, None], dnums, (1,),
                      mode=lax.GatherScatterMode.PROMISE_IN_BOUNDS)


def _embed_call(x0p, x1p, dpp, type3, attr3, depth3):
    @functools.partial(
        pl.kernel,
        out_type=jax.ShapeDtypeStruct((NEMB, 8, 16), jnp.float32),
        mesh=_mesh(),
        compiler_params=_SC_PARAMS,
        scratch_types=[
            pltpu.VMEM((EMB_BATCH,), jnp.int32),
            pltpu.VMEM((EMB_BATCH,), jnp.int32),
            pltpu.VMEM((EMB_BATCH,), jnp.int32),
            pltpu.VMEM((EMB_BATCH, 8, 16), jnp.float32),
            pltpu.VMEM((EMB_BATCH, 8, 16), jnp.float32),
            pltpu.VMEM((EMB_BATCH, 8, 16), jnp.float32),
            pltpu.SemaphoreType.DMA,
        ],
    )
    def k(x0_h, x1_h, dp_h, t_h, a_h, d_h, out_h, i0, i1, i2, tb, ab, db, sem):
        c = lax.axis_index("c")
        s = lax.axis_index("s")
        wid = c * NS + s
        base = wid * EMB_ROWS_PER_W

        def step(it, _):
            rb = base + it * EMB_BATCH
            pltpu.sync_copy(x0_h.at[pl.ds(rb, EMB_BATCH)], i0)
            pltpu.sync_copy(x1_h.at[pl.ds(rb, EMB_BATCH)], i1)
            pltpu.sync_copy(dp_h.at[pl.ds(rb, EMB_BATCH)], i2)
            for kk in range(EMB_BATCH // 16):
                sl = pl.ds(kk * 16, 16)
                i2[sl] = jnp.minimum(i2[sl], MAX_DEPTH)
            cp0 = pltpu.async_copy(t_h.at[i0], tb, sem)
            cp1 = pltpu.async_copy(a_h.at[i1], ab, sem)
            cp2 = pltpu.async_copy(d_h.at[i2], db, sem)
            cp0.wait()
            cp1.wait()
            cp2.wait()

            def row(r, _):
                for j in range(8):
                    tb[r, j] = tb[r, j] + ab[r, j] + db[r, j]
                return 0

            lax.fori_loop(0, EMB_BATCH, row, 0)
            pltpu.sync_copy(tb, out_h.at[pl.ds(rb, EMB_BATCH)])
            return 0

        lax.fori_loop(0, EMB_ITERS, step, 0)

    return k(x0p, x1p, dpp, type3, attr3, depth3)


def _pre_call(h, W, asv, adv, C):
    def body(h_ref, w_ref, as_ref, ad_ref, hw_ref, t_ref, m_ref):
        hw = jnp.dot(h_ref[...], w_ref[...], preferred_element_type=jnp.float32)
        hw_p = jnp.concatenate(
            [hw, jnp.zeros((NPAD - N, HID), jnp.float32)], axis=0)
        hw_ref[...] = hw_p
        kio = lax.broadcasted_iota(jnp.int32, (HID, 16), 0)
        jio = lax.broadcasted_iota(jnp.int32, (HID, 16), 1)
        head = kio // C
        S = (jnp.where(jio == head, as_ref[...], 0.0)
             + jnp.where(jio == head + 8, ad_ref[...], 0.0))
        T = jnp.dot(hw_p, S, preferred_element_type=jnp.float32)
        t_ref[...] = T
        colmax = jnp.max(T, axis=0, keepdims=True)
        a8 = colmax[:, 0:8]
        d8 = colmax[:, 8:16]
        b8 = a8 + d8
        b8 = jnp.where(b8 > 0, b8, 0.2 * b8)
        mrow = jnp.concatenate([b8, jnp.zeros((1, 8), jnp.float32)], axis=1)
        m_ref[...] = jnp.broadcast_to(mrow, (8, 16))

    return pl.pallas_call(
        body,
        out_shape=(
            jax.ShapeDtypeStruct((NPAD, HID), jnp.float32),
            jax.ShapeDtypeStruct((NPAD, 16), jnp.float32),
            jax.ShapeDtypeStruct((8, 16), jnp.float32),
        ),
    )(h, W, asv, adv)


def _edge_a_call(T, M, srcp, dstp, z16, H, nbatch):
    etot = srcp.shape[0]
    epw = etot // NW

    @functools.partial(
        pl.kernel,
        out_type=(
            jax.ShapeDtypeStruct((NPAD, 16), jnp.float32),
            jax.ShapeDtypeStruct((NPAD, 16), jnp.float32),
            jax.ShapeDtypeStruct((etot, 16), jnp.float32),
        ),
        mesh=_mesh(),
        compiler_params=_SC_PARAMS,
        scratch_types=[
            pltpu.VMEM((EB,), jnp.int32),
            pltpu.VMEM((EB,), jnp.int32),
            pltpu.VMEM((EB, 16), jnp.float32),
            pltpu.VMEM((EB, 16), jnp.float32),
            pltpu.VMEM((EB, 16), jnp.float32),
            pltpu.VMEM((16,), jnp.float32),
            pltpu.VMEM_SHARED((NPAD, 16), jnp.float32),
            pltpu.SemaphoreType.DMA,
        ],
    )
    def k(t_h, m_h, src_h, dst_h, z_h, den0_h, den1_h, ex_h,
          sidx, didx, tsrc, tdst, exb, mb, dacc, sem):
        c = lax.axis_index("c")
        s = lax.axis_index("s")
        base = (c * NS + s) * epw
        rsl = pl.ds(s * ROWS_PER_TILE, ROWS_PER_TILE)
        pltpu.sync_copy(z_h.at[rsl], dacc.at[rsl])
        pltpu.sync_copy(m_h.at[0], mb)
        plsc.subcore_barrier()

        lane = lax.iota(jnp.int32, 16)
        perm = jnp.where(lane < 8, lane + 8, lane - 8)
        mv = mb[...]

        def batch(b, _):
            eb = base + b * EB
            pltpu.sync_copy(src_h.at[pl.ds(eb, EB)], sidx)
            pltpu.sync_copy(dst_h.at[pl.ds(eb, EB)], didx)
            cp0 = pltpu.async_copy(t_h.at[sidx], tsrc, sem)
            cp1 = pltpu.async_copy(t_h.at[didx], tdst, sem)
            cp0.wait()
            cp1.wait()

            def edge(e, _):
                av = tsrc[e]
                dv = _take16(tdst[e], perm)
                ev = av + dv
                lr = jnp.where(ev > 0, ev, 0.2 * ev)
                exv = jnp.exp(lr - mv)
                exb[e] = jnp.where(lane < H, exv, 0.0)
                return 0

            lax.fori_loop(0, EB, edge, 0)
            pltpu.sync_copy(exb, ex_h.at[pl.ds(eb, EB)])
            pltpu.sync_copy(exb, dacc.at[didx], add=True)
            return 0

        lax.fori_loop(0, nbatch, batch, 0)
        plsc.subcore_barrier()

        @pl.when(c == 0)
        def _():
            pltpu.sync_copy(dacc.at[rsl], den0_h.at[rsl])

        @pl.when(c == 1)
        def _():
            pltpu.sync_copy(dacc.at[rsl], den1_h.at[rsl])

    return k(T, M, srcp, dstp, z16)


def _edge_b_call(hw3, exall, den0, den1, srcp, dstp, z128, H, nbatch):
    etot = srcp.shape[0]
    epw = etot // NW
    C = HID // H

    @functools.partial(
        pl.kernel,
        out_type=(
            jax.ShapeDtypeStruct((NPAD, 8, 16), jnp.float32),
            jax.ShapeDtypeStruct((NPAD, 8, 16), jnp.float32),
        ),
        mesh=_mesh(),
        compiler_params=_SC_PARAMS,
        scratch_types=[
            pltpu.VMEM((EB,), jnp.int32),
            pltpu.VMEM((EB,), jnp.int32),
            pltpu.VMEM((EB, 8, 16), jnp.float32),
            pltpu.VMEM((EB, 16), jnp.float32),
            pltpu.VMEM((EB, 16), jnp.float32),
            pltpu.VMEM((EB, 16), jnp.float32),
            pltpu.VMEM_SHARED((NPAD, 8, 16), jnp.float32),
            pltpu.SemaphoreType.DMA,
        ],
    )
    def k(hw_h, ex_h, d0_h, d1_h, src_h, dst_h, z_h, o0_h, o1_h,
          sidx, didx, hb, exb, d0b, d1b, oacc, sem):
        c = lax.axis_index("c")
        s = lax.axis_index("s")
        base = (c * NS + s) * epw
        rsl = pl.ds(s * ROWS_PER_TILE, ROWS_PER_TILE)
        pltpu.sync_copy(z_h.at[rsl], oacc.at[rsl])
        plsc.subcore_barrier()

        heads = sorted(set((j * 16) // C for j in range(8)))

        def batch(b, _):
            eb = base + b * EB
            pltpu.sync_copy(src_h.at[pl.ds(eb, EB)], sidx)
            pltpu.sync_copy(dst_h.at[pl.ds(eb, EB)], didx)
            cp0 = pltpu.async_copy(hw_h.at[sidx], hb, sem)
            cp1 = pltpu.async_copy(d0_h.at[didx], d0b, sem)
            cp2 = pltpu.async_copy(d1_h.at[didx], d1b, sem)
            pltpu.sync_copy(ex_h.at[pl.ds(eb, EB)], exb)
            cp0.wait()
            cp1.wait()
            cp2.wait()

            def edge(e, _):
                den = d0b[e] + d1b[e] + 1e-16
                alv = exb[e] / den
                bcast = {}
                for hd in heads:
                    bcast[hd] = _take16(
                        alv, jnp.full((16,), hd, jnp.int32))
                for j in range(8):
                    hd = (j * 16) // C
                    hb[e, j] = hb[e, j] * bcast[hd]
                return 0

            lax.fori_loop(0, EB, edge, 0)
            pltpu.sync_copy(hb, oacc.at[didx], add=True)
            return 0

        lax.fori_loop(0, nbatch, batch, 0)
        plsc.subcore_barrier()

        @pl.when(c == 0)
        def _():
            pltpu.sync_copy(oacc.at[rsl], o0_h.at[rsl])

        @pl.when(c == 1)
        def _():
            pltpu.sync_copy(oacc.at[rsl], o1_h.at[rsl])

    return k(hw3, exall, den0, den1, srcp, dstp, z128)


def _post_call(o0f, o1f, bvec, gvec, bevec, h_in):
    def body(o0_ref, o1_ref, b_ref, g_ref, be_ref, h_ref, out_ref):
        x = o0_ref[0:N, :] + o1_ref[0:N, :] + b_ref[...]
        mu = jnp.mean(x, axis=0, keepdims=True)
        d = x - mu
        var = jnp.mean(d * d, axis=0, keepdims=True)
        xn = d / jnp.sqrt(var + 1e-5) * g_ref[...] + be_ref[...]
        out_ref[...] = jnp.maximum(xn, 0.0) + h_ref[...]

    return pl.pallas_call(
        body,
        out_shape=jax.ShapeDtypeStruct((N, HID), jnp.float32),
    )(o0f, o1f, bvec, gvec, bevec, h_in)


def _pool_call(h, batchf):
    def body(h_ref, b_ref, out_ref):
        gid = lax.broadcasted_iota(jnp.int32, (N, NGRAPHS), 1)
        oh = jnp.where(b_ref[...] == gid, 1.0, 0.0)
        sums_t = lax.dot_general(
            h_ref[...], oh, (((0,), (0,)), ((), ())),
            preferred_element_type=jnp.float32)
        cnts = jnp.sum(oh, axis=0, keepdims=True)
        pooled_t = sums_t / jnp.maximum(cnts, 1.0)
        out_ref[...] = pooled_t.T

    return pl.pallas_call(
        body,
        out_shape=jax.ShapeDtypeStruct((NGRAPHS, HID), jnp.float32),
    )(h, batchf)


def _head_call(pooled, token_W, token_b):
    SEQ, _, V2 = token_W.shape

    def body(p_ref, w_ref, b_ref, out_ref):
        out_ref[0] = (jnp.dot(p_ref[...], w_ref[0],
                              preferred_element_type=jnp.float32)
                      + b_ref[0])

    return pl.pallas_call(
        body,
        grid=(SEQ,),
        in_specs=[
            pl.BlockSpec((NGRAPHS, HID), lambda s: (0, 0)),
            pl.BlockSpec((1, HID, V2), lambda s: (s, 0, 0)),
            pl.BlockSpec((1, 1, V2), lambda s: (s, 0, 0)),
        ],
        out_specs=pl.BlockSpec((1, NGRAPHS, V2), lambda s: (s, 0, 0)),
        out_shape=jax.ShapeDtypeStruct((SEQ, NGRAPHS, V2), jnp.float32),
    )(pooled, token_W, token_b)


def kernel(x, node_depth, edge_index, batch, type_emb, attr_emb, depth_emb,
           W0, as0, ad0, b0, g0, be0, W1, as1, ad1, b1, g1, be1,
           W2, as2, ad2, b2, g2, be2, W3, as3, ad3, b3, g3, be3,
           token_W, token_b):
    E = edge_index.shape[1]
    etot = E + N
    etot_pad = ((etot + NW * EB - 1) // (NW * EB)) * (NW * EB)
    nbatch = etot_pad // (NW * EB)

    x0p = jnp.concatenate(
        [x[:, 0].astype(jnp.int32), jnp.zeros((NEMB - N,), jnp.int32)])
    x1p = jnp.concatenate(
        [x[:, 1].astype(jnp.int32), jnp.zeros((NEMB - N,), jnp.int32)])
    dpp = jnp.concatenate(
        [node_depth.reshape(-1).astype(jnp.int32),
         jnp.zeros((NEMB - N,), jnp.int32)])
    loops = jnp.arange(N, dtype=jnp.int32)
    pad_e = jnp.full((etot_pad - etot,), N, jnp.int32)
    srcp = jnp.concatenate([edge_index[0].astype(jnp.int32), loops, pad_e])
    dstp = jnp.concatenate([edge_index[1].astype(jnp.int32), loops, pad_e])

    type3 = type_emb.reshape(-1, 8, 16)
    attr3 = attr_emb.reshape(-1, 8, 16)
    depth3 = depth_emb.reshape(-1, 8, 16)
    z16 = jnp.zeros((NPAD, 16), jnp.float32)
    z128 = jnp.zeros((NPAD, 8, 16), jnp.float32)
    batchf = batch.astype(jnp.int32).reshape(N, 1)

    h0 = _embed_call(x0p, x1p, dpp, type3, attr3, depth3)
    h = h0.reshape(NEMB, HID)[:N]

    params = [(W0, as0, ad0, b0, g0, be0), (W1, as1, ad1, b1, g1, be1),
              (W2, as2, ad2, b2, g2, be2), (W3, as3, ad3, b3, g3, be3)]
    for (W, a_s, a_d, b, g, be), H in zip(params, HEADS):
        C = HID // H
        asv = a_s.reshape(HID, 1)
        adv = a_d.reshape(HID, 1)
        hw, T, M = _pre_call(h, W, asv, adv, C)
        den0, den1, exall = _edge_a_call(T, M, srcp, dstp, z16, H, nbatch)
        hw3 = hw.reshape(NPAD, 8, 16)
        o0, o1 = _edge_b_call(hw3, exall, den0, den1, srcp, dstp, z128,
                              H, nbatch)
        h = _post_call(o0.reshape(NPAD, HID), o1.reshape(NPAD, HID),
                       b.reshape(1, HID), g.reshape(1, HID),
                       be.reshape(1, HID), h)

    pooled = _pool_call(h, batchf)
    return _head_call(pooled, token_W, token_b.reshape(token_b.shape[0], 1, -1))


# speedup vs baseline: 38.9299x; 1.0713x over previous
import functools

import jax
import jax.numpy as jnp
from jax import lax
from jax.experimental import pallas as pl
from jax.experimental.pallas import tpu as pltpu
from jax.experimental.pallas import tpu_sc as plsc

N = 10000
HID = 128
NGRAPHS = 128
MAX_DEPTH = 20
HEADS = (8, 8, 8, 1)

NC = 2
NS = 16
NW = NC * NS

NPAD = 10016
ROWS_PER_TILE = NPAD // NS

NEMB = 10240
EMB_ROWS_PER_W = NEMB // NW
EMB_BATCH = 80
EMB_ITERS = EMB_ROWS_PER_W // EMB_BATCH

EB = 128


def _mesh():
    return plsc.VectorSubcoreMesh(core_axis_name="c", subcore_axis_name="s")


_SC_PARAMS = pltpu.CompilerParams(use_tc_tiling_on_sc=False)


def _take16(x, idx):
    dnums = lax.GatherDimensionNumbers(
        offset_dims=(), collapsed_slice_dims=(0,), start_index_map=(0,))
    return lax.gather(x, idx[:, None], dnums, (1,),
                      mode=lax.GatherScatterMode.PROMISE_IN_BOUNDS)


def _embed_call(x0p, x1p, dpp, type3, attr3, depth3):
    @functools.partial(
        pl.kernel,
        out_type=jax.ShapeDtypeStruct((NEMB, 8, 16), jnp.float32),
        mesh=_mesh(),
        compiler_params=_SC_PARAMS,
        scratch_types=[
            pltpu.VMEM((EMB_BATCH,), jnp.int32),
            pltpu.VMEM((EMB_BATCH,), jnp.int32),
            pltpu.VMEM((EMB_BATCH,), jnp.int32),
            pltpu.VMEM((EMB_BATCH, 8, 16), jnp.float32),
            pltpu.VMEM((EMB_BATCH, 8, 16), jnp.float32),
            pltpu.VMEM((EMB_BATCH, 8, 16), jnp.float32),
            pltpu.SemaphoreType.DMA,
        ],
    )
    def k(x0_h, x1_h, dp_h, t_h, a_h, d_h, out_h, i0, i1, i2, tb, ab, db, sem):
        c = lax.axis_index("c")
        s = lax.axis_index("s")
        wid = c * NS + s
        base = wid * EMB_ROWS_PER_W

        def step(it, _):
            rb = base + it * EMB_BATCH
            pltpu.sync_copy(x0_h.at[pl.ds(rb, EMB_BATCH)], i0)
            pltpu.sync_copy(x1_h.at[pl.ds(rb, EMB_BATCH)], i1)
            pltpu.sync_copy(dp_h.at[pl.ds(rb, EMB_BATCH)], i2)
            for kk in range(EMB_BATCH // 16):
                sl = pl.ds(kk * 16, 16)
                i2[sl] = jnp.minimum(i2[sl], MAX_DEPTH)
            cp0 = pltpu.async_copy(t_h.at[i0], tb, sem)
            cp1 = pltpu.async_copy(a_h.at[i1], ab, sem)
            cp2 = pltpu.async_copy(d_h.at[i2], db, sem)
            cp0.wait()
            cp1.wait()
            cp2.wait()

            def row(r, _):
                for j in range(8):
                    tb[r, j] = tb[r, j] + ab[r, j] + db[r, j]
                return 0

            lax.fori_loop(0, EMB_BATCH, row, 0)
            pltpu.sync_copy(tb, out_h.at[pl.ds(rb, EMB_BATCH)])
            return 0

        lax.fori_loop(0, EMB_ITERS, step, 0)

    return k(x0p, x1p, dpp, type3, attr3, depth3)


def _pre_call(h, W, asv, adv, C):
    def body(h_ref, w_ref, as_ref, ad_ref, hw_ref, t_ref, m_ref):
        hw = jnp.dot(h_ref[...], w_ref[...], preferred_element_type=jnp.float32)
        hw_p = jnp.concatenate(
            [hw, jnp.zeros((NPAD - N, HID), jnp.float32)], axis=0)
        hw_ref[...] = hw_p
        kio = lax.broadcasted_iota(jnp.int32, (HID, 16), 0)
        jio = lax.broadcasted_iota(jnp.int32, (HID, 16), 1)
        head = kio // C
        S = (jnp.where(jio == head, as_ref[...], 0.0)
             + jnp.where(jio == head + 8, ad_ref[...], 0.0))
        T = jnp.dot(hw_p, S, preferred_element_type=jnp.float32)
        t_ref[...] = T
        colmax = jnp.max(T, axis=0, keepdims=True)
        a8 = colmax[:, 0:8]
        d8 = colmax[:, 8:16]
        b8 = a8 + d8
        b8 = jnp.where(b8 > 0, b8, 0.2 * b8)
        mrow = jnp.concatenate([b8, jnp.zeros((1, 8), jnp.float32)], axis=1)
        m_ref[...] = jnp.broadcast_to(mrow, (8, 16))

    return pl.pallas_call(
        body,
        out_shape=(
            jax.ShapeDtypeStruct((NPAD, HID), jnp.float32),
            jax.ShapeDtypeStruct((NPAD, 16), jnp.float32),
            jax.ShapeDtypeStruct((8, 16), jnp.float32),
        ),
    )(h, W, asv, adv)


def _edge_a_call(T, M, srcp, dstp, z16, H, nbatch):
    etot = srcp.shape[0]
    epw = etot // NW

    @functools.partial(
        pl.kernel,
        out_type=(
            jax.ShapeDtypeStruct((NPAD, 16), jnp.float32),
            jax.ShapeDtypeStruct((NPAD, 16), jnp.float32),
            jax.ShapeDtypeStruct((etot, 16), jnp.float32),
        ),
        mesh=_mesh(),
        compiler_params=_SC_PARAMS,
        scratch_types=[
            [pltpu.VMEM((EB,), jnp.int32)] * 2,
            [pltpu.VMEM((EB,), jnp.int32)] * 2,
            [pltpu.VMEM((EB, 16), jnp.float32)] * 2,
            [pltpu.VMEM((EB, 16), jnp.float32)] * 2,
            [pltpu.VMEM((EB, 16), jnp.float32)] * 2,
            pltpu.VMEM((16,), jnp.float32),
            pltpu.VMEM_SHARED((NPAD, 16), jnp.float32),
            [pltpu.SemaphoreType.DMA] * 2,
        ],
    )
    def k(t_h, m_h, src_h, dst_h, z_h, den0_h, den1_h, ex_h,
          sidx, didx, tsrc, tdst, exb, mb, dacc, sem):
        c = lax.axis_index("c")
        s = lax.axis_index("s")
        base = (c * NS + s) * epw
        rsl = pl.ds(s * ROWS_PER_TILE, ROWS_PER_TILE)
        pltpu.sync_copy(z_h.at[rsl], dacc.at[rsl])
        pltpu.sync_copy(m_h.at[0], mb)
        plsc.subcore_barrier()

        lane = lax.iota(jnp.int32, 16)
        perm = jnp.where(lane < 8, lane + 8, lane - 8)
        mv = mb[...]

        def stage(p, eb):
            pltpu.sync_copy(src_h.at[pl.ds(eb, EB)], sidx[p])
            pltpu.sync_copy(dst_h.at[pl.ds(eb, EB)], didx[p])
            cp0 = pltpu.async_copy(t_h.at[sidx[p]], tsrc[p], sem[p])
            cp1 = pltpu.async_copy(t_h.at[didx[p]], tdst[p], sem[p])
            return cp0, cp1

        def run(p, eb, cps):
            cps[0].wait()
            cps[1].wait()

            @plsc.parallel_loop(0, EB, unroll=4)
            def _(e):
                av = tsrc[p][e]
                dv = _take16(tdst[p][e], perm)
                ev = av + dv
                lr = jnp.where(ev > 0, ev, 0.2 * ev)
                exv = jnp.exp(lr - mv)
                exb[p][e] = jnp.where(lane < H, exv, 0.0)

            pltpu.sync_copy(exb[p], ex_h.at[pl.ds(eb, EB)])
            pltpu.sync_copy(exb[p], dacc.at[didx[p]], add=True)

        def pairs(i, _):
            eb0 = base + (2 * i) * EB
            eb1 = eb0 + EB
            cpa = stage(0, eb0)
            cpb = stage(1, eb1)
            run(0, eb0, cpa)
            run(1, eb1, cpb)
            return 0

        lax.fori_loop(0, nbatch // 2, pairs, 0)
        plsc.subcore_barrier()

        @pl.when(c == 0)
        def _():
            pltpu.sync_copy(dacc.at[rsl], den0_h.at[rsl])

        @pl.when(c == 1)
        def _():
            pltpu.sync_copy(dacc.at[rsl], den1_h.at[rsl])

    return k(T, M, srcp, dstp, z16)


def _edge_b_call(hw3, exall, srcp, dstp, z128, H, nbatch):
    etot = srcp.shape[0]
    epw = etot // NW
    C = HID // H
    heads = sorted(set((j * 16) // C for j in range(8)))

    @functools.partial(
        pl.kernel,
        out_type=(
            jax.ShapeDtypeStruct((NPAD, 8, 16), jnp.float32),
            jax.ShapeDtypeStruct((NPAD, 8, 16), jnp.float32),
        ),
        mesh=_mesh(),
        compiler_params=_SC_PARAMS,
        scratch_types=[
            [pltpu.VMEM((EB,), jnp.int32)] * 2,
            [pltpu.VMEM((EB,), jnp.int32)] * 2,
            [pltpu.VMEM((EB, 8, 16), jnp.float32)] * 2,
            [pltpu.VMEM((EB, 16), jnp.float32)] * 2,
            pltpu.VMEM_SHARED((NPAD, 8, 16), jnp.float32),
            [pltpu.SemaphoreType.DMA] * 2,
        ],
    )
    def k(hw_h, ex_h, src_h, dst_h, z_h, o0_h, o1_h,
          sidx, didx, hb, exb, oacc, sem):
        c = lax.axis_index("c")
        s = lax.axis_index("s")
        base = (c * NS + s) * epw
        rsl = pl.ds(s * ROWS_PER_TILE, ROWS_PER_TILE)
        pltpu.sync_copy(z_h.at[rsl], oacc.at[rsl])
        plsc.subcore_barrier()

        def stage(p, eb):
            pltpu.sync_copy(src_h.at[pl.ds(eb, EB)], sidx[p])
            pltpu.sync_copy(dst_h.at[pl.ds(eb, EB)], didx[p])
            cp = pltpu.async_copy(hw_h.at[sidx[p]], hb[p], sem[p])
            pltpu.sync_copy(ex_h.at[pl.ds(eb, EB)], exb[p])
            return cp

        def run(p, cp):
            cp.wait()

            @plsc.parallel_loop(0, EB, unroll=4)
            def _(e):
                alv = exb[p][e]
                bcast = {hd: _take16(alv, jnp.full((16,), hd, jnp.int32))
                         for hd in heads}
                for j in range(8):
                    hd = (j * 16) // C
                    hb[p][e, j] = hb[p][e, j] * bcast[hd]

            pltpu.sync_copy(hb[p], oacc.at[didx[p]], add=True)

        def pairs(i, _):
            eb0 = base + (2 * i) * EB
            eb1 = eb0 + EB
            cpa = stage(0, eb0)
            cpb = stage(1, eb1)
            run(0, cpa)
            run(1, cpb)
            return 0

        lax.fori_loop(0, nbatch // 2, pairs, 0)
        plsc.subcore_barrier()

        @pl.when(c == 0)
        def _():
            pltpu.sync_copy(oacc.at[rsl], o0_h.at[rsl])

        @pl.when(c == 1)
        def _():
            pltpu.sync_copy(oacc.at[rsl], o1_h.at[rsl])

    return k(hw3, exall, srcp, dstp, z128)


def _post_call(o0f, o1f, den0, den1, bvec, gvec, bevec, h_in, C):
    def body(o0_ref, o1_ref, dn0_ref, dn1_ref, b_ref, g_ref, be_ref, h_ref,
             out_ref):
        u = o0_ref[0:N, :] + o1_ref[0:N, :]
        den = dn0_ref[0:N, :] + dn1_ref[0:N, :] + 1e-16
        lio = lax.broadcasted_iota(jnp.int32, (16, HID), 0)
        kio = lax.broadcasted_iota(jnp.int32, (16, HID), 1)
        R = jnp.where(lio == kio // C, 1.0, 0.0)
        denex = jnp.dot(den, R, preferred_element_type=jnp.float32)
        x = u / denex + b_ref[...]
        mu = jnp.mean(x, axis=0, keepdims=True)
        d = x - mu
        var = jnp.mean(d * d, axis=0, keepdims=True)
        xn = d / jnp.sqrt(var + 1e-5) * g_ref[...] + be_ref[...]
        out_ref[...] = jnp.maximum(xn, 0.0) + h_ref[...]

    return pl.pallas_call(
        body,
        out_shape=jax.ShapeDtypeStruct((N, HID), jnp.float32),
    )(o0f, o1f, den0, den1, bvec, gvec, bevec, h_in)


def _pool_call(h, batchf):
    def body(h_ref, b_ref, out_ref):
        gid = lax.broadcasted_iota(jnp.int32, (N, NGRAPHS), 1)
        oh = jnp.where(b_ref[...] == gid, 1.0, 0.0)
        sums_t = lax.dot_general(
            h_ref[...], oh, (((0,), (0,)), ((), ())),
            preferred_element_type=jnp.float32)
        cnts = jnp.sum(oh, axis=0, keepdims=True)
        pooled_t = sums_t / jnp.maximum(cnts, 1.0)
        out_ref[...] = pooled_t.T

    return pl.pallas_call(
        body,
        out_shape=jax.ShapeDtypeStruct((NGRAPHS, HID), jnp.float32),
    )(h, batchf)


def _head_call(pooled, token_W, token_b):
    SEQ, _, V2 = token_W.shape

    def body(p_ref, w_ref, b_ref, out_ref):
        out_ref[0] = (jnp.dot(p_ref[...], w_ref[0],
                              preferred_element_type=jnp.float32)
                      + b_ref[0])

    return pl.pallas_call(
        body,
        grid=(SEQ,),
        in_specs=[
            pl.BlockSpec((NGRAPHS, HID), lambda s: (0, 0)),
            pl.BlockSpec((1, HID, V2), lambda s: (s, 0, 0)),
            pl.BlockSpec((1, 1, V2), lambda s: (s, 0, 0)),
        ],
        out_specs=pl.BlockSpec((1, NGRAPHS, V2), lambda s: (s, 0, 0)),
        out_shape=jax.ShapeDtypeStruct((SEQ, NGRAPHS, V2), jnp.float32),
    )(pooled, token_W, token_b)


def kernel(x, node_depth, edge_index, batch, type_emb, attr_emb, depth_emb,
           W0, as0, ad0, b0, g0, be0, W1, as1, ad1, b1, g1, be1,
           W2, as2, ad2, b2, g2, be2, W3, as3, ad3, b3, g3, be3,
           token_W, token_b):
    E = edge_index.shape[1]
    etot = E + N
    etot_pad = ((etot + 2 * NW * EB - 1) // (2 * NW * EB)) * (2 * NW * EB)
    nbatch = etot_pad // (NW * EB)

    x0p = jnp.concatenate(
        [x[:, 0].astype(jnp.int32), jnp.zeros((NEMB - N,), jnp.int32)])
    x1p = jnp.concatenate(
        [x[:, 1].astype(jnp.int32), jnp.zeros((NEMB - N,), jnp.int32)])
    dpp = jnp.concatenate(
        [node_depth.reshape(-1).astype(jnp.int32),
         jnp.zeros((NEMB - N,), jnp.int32)])
    loops = jnp.arange(N, dtype=jnp.int32)
    pad_e = jnp.full((etot_pad - etot,), N, jnp.int32)
    srcp = jnp.concatenate([edge_index[0].astype(jnp.int32), loops, pad_e])
    dstp = jnp.concatenate([edge_index[1].astype(jnp.int32), loops, pad_e])

    type3 = type_emb.reshape(-1, 8, 16)
    attr3 = attr_emb.reshape(-1, 8, 16)
    depth3 = depth_emb.reshape(-1, 8, 16)
    z16 = jnp.zeros((NPAD, 16), jnp.float32)
    z128 = jnp.zeros((NPAD, 8, 16), jnp.float32)
    batchf = batch.astype(jnp.int32).reshape(N, 1)

    h0 = _embed_call(x0p, x1p, dpp, type3, attr3, depth3)
    h = h0.reshape(NEMB, HID)[:N]

    params = [(W0, as0, ad0, b0, g0, be0), (W1, as1, ad1, b1, g1, be1),
              (W2, as2, ad2, b2, g2, be2), (W3, as3, ad3, b3, g3, be3)]
    for (W, a_s, a_d, b, g, be), H in zip(params, HEADS):
        C = HID // H
        asv = a_s.reshape(HID, 1)
        adv = a_d.reshape(HID, 1)
        hw, T, M = _pre_call(h, W, asv, adv, C)
        den0, den1, exall = _edge_a_call(T, M, srcp, dstp, z16, H, nbatch)
        hw3 = hw.reshape(NPAD, 8, 16)
        o0, o1 = _edge_b_call(hw3, exall, srcp, dstp, z128, H, nbatch)
        h = _post_call(o0.reshape(NPAD, HID), o1.reshape(NPAD, HID),
                       den0, den1, b.reshape(1, HID), g.reshape(1, HID),
                       be.reshape(1, HID), h, C)

    pooled = _pool_call(h, batchf)
    return _head_call(pooled, token_W, token_b.reshape(token_b.shape[0], 1, -1))
